# baseline (speedup 1.0000x reference)
import jax
import jax.numpy as jnp
from jax.experimental import pallas as pl

B = 256
T = 16
F = 16080
C = 1000
K = 200
BBLK = 16


def _body(mask_ref, x_ref, out_ref, att_ref):
    @pl.when(pl.program_id(0) == 0)
    def _compute_attention():
        s = jax.nn.sigmoid(mask_ref[...])
        bits = jax.lax.bitcast_convert_type(s, jnp.int32)

        def bstep(_, lohi):
            lo, hi = lohi
            mid = lo + (hi - lo + 1) // 2
            cnt = jnp.sum((bits >= mid).astype(jnp.int32), axis=1, keepdims=True)
            ge = cnt >= K
            return jnp.where(ge, mid, lo), jnp.where(ge, hi, mid - 1)

        lo0 = jnp.zeros((T, 1), jnp.int32)
        hi0 = jnp.full((T, 1), 0x3F800000, jnp.int32)
        thr, _ = jax.lax.fori_loop(0, 31, bstep, (lo0, hi0))

        col = jax.lax.broadcasted_iota(jnp.int32, (T, C), 1)
        gt = bits > thr
        eq = bits == thr
        need = K - jnp.sum(gt.astype(jnp.int32), axis=1, keepdims=True)

        def istep(_, lohi):
            lo, hi = lohi
            mid = (lo + hi) // 2
            cnt = jnp.sum((eq & (col < mid)).astype(jnp.int32), axis=1,
                          keepdims=True)
            ok = cnt >= need
            return jnp.where(ok, lo, mid + 1), jnp.where(ok, mid, hi)

        plo0 = jnp.zeros((T, 1), jnp.int32)
        phi0 = jnp.full((T, 1), C, jnp.int32)
        pcut, _ = jax.lax.fori_loop(0, 10, istep, (plo0, phi0))

        keep = gt | (eq & (col < pcut))
        att_ref[:, :C] = jnp.where(keep, s, 0.0)
        att_ref[:, C:] = jnp.zeros((T, F - C), jnp.float32)

    out_ref[...] = jnp.zeros((BBLK, T, F), jnp.float32)


def kernel(x, attention_mask):
    x2 = x.reshape(B, F)
    out, att = pl.pallas_call(
        _body,
        grid=(B // BBLK,),
        in_specs=[
            pl.BlockSpec((T, C), lambda i: (0, 0)),
            pl.BlockSpec((BBLK, F), lambda i: (i, 0)),
        ],
        out_specs=[
            pl.BlockSpec((BBLK, T, F), lambda i: (i, 0, 0)),
            pl.BlockSpec((T, F), lambda i: (0, 0)),
        ],
        out_shape=[
            jax.ShapeDtypeStruct((B, T, F), jnp.float32),
            jax.ShapeDtypeStruct((T, F), jnp.float32),
        ],
    )(attention_mask, x2)
    return out, att


# speedup vs baseline: 1.0077x; 1.0077x over previous
import jax
import jax.numpy as jnp
from jax.experimental import pallas as pl
from jax.experimental.pallas import tpu as pltpu

B = 256
T = 16
F = 16080
C = 1000
CP = 1024
K = 200

NZQ = 8
ZROWS = 8
NDQ = 2
DROWS = 32


def _attention_values(mask):
    s = jax.nn.sigmoid(mask)
    bits = jax.lax.bitcast_convert_type(s, jnp.int32)

    def bstep(_, lohi):
        lo, hi = lohi
        mid = lo + (hi - lo + 1) // 2
        cnt = jnp.sum((bits >= mid).astype(jnp.int32), axis=1, keepdims=True)
        ge = cnt >= K
        return jnp.where(ge, mid, lo), jnp.where(ge, hi, mid - 1)

    lo0 = jnp.zeros((T, 1), jnp.int32)
    hi0 = jnp.full((T, 1), 0x3F800000, jnp.int32)
    thr, _ = jax.lax.fori_loop(0, 31, bstep, (lo0, hi0))

    col = jax.lax.broadcasted_iota(jnp.int32, (T, C), 1)
    gt = bits > thr
    eq = bits == thr
    need = K - jnp.sum(gt.astype(jnp.int32), axis=1, keepdims=True)

    def istep(_, lohi):
        lo, hi = lohi
        mid = (lo + hi) // 2
        cnt = jnp.sum((eq & (col < mid)).astype(jnp.int32), axis=1,
                      keepdims=True)
        ok = cnt >= need
        return jnp.where(ok, lo, mid + 1), jnp.where(ok, mid, hi)

    plo0 = jnp.zeros((T, 1), jnp.int32)
    phi0 = jnp.full((T, 1), C, jnp.int32)
    pcut, _ = jax.lax.fori_loop(0, 10, istep, (plo0, phi0))

    keep = gt | (eq & (col < pcut))
    return jnp.where(keep, s, 0.0)


def _body(mask_ref, x_ref, out_ref, att_ref, zbuf, dbufs, zsems, dsems):
    att = _attention_values(mask_ref[...])
    att_ref[:, :C] = att
    att_ref[:, C:] = jnp.zeros((T, F - C), jnp.float32)

    zbuf[...] = jnp.zeros((ZROWS, T, F - CP), jnp.float32)

    def zcopy(i):
        return pltpu.make_async_copy(
            zbuf,
            out_ref.at[pl.ds(i * ZROWS, ZROWS), :, pl.ds(CP, F - CP)],
            zsems.at[i % NZQ],
        )

    nz = B // ZROWS
    for i in range(nz):
        if i >= NZQ:
            zcopy(i - NZQ).wait()
        zcopy(i).start()

    attp = jnp.concatenate(
        [att, jnp.zeros((T, CP - C), jnp.float32)], axis=1)

    def dcopy(j, buf):
        return pltpu.make_async_copy(
            buf,
            out_ref.at[pl.ds(j * DROWS, DROWS), :, pl.ds(0, CP)],
            dsems.at[j % NDQ],
        )

    nd = B // DROWS
    for j in range(nd):
        buf = dbufs[j % NDQ]
        if j >= NDQ:
            dcopy(j - NDQ, buf).wait()
        xs = x_ref[pl.ds(j * DROWS, DROWS), :]
        buf[...] = xs[:, None, :] * attp[None, :, :]
        dcopy(j, buf).start()

    for i in range(nz - NZQ, nz):
        zcopy(i).wait()
    for j in range(nd - NDQ, nd):
        dcopy(j, dbufs[j % NDQ]).wait()


def kernel(x, attention_mask):
    x_head = x.reshape(B, F)[:, :CP]
    out, att = pl.pallas_call(
        _body,
        in_specs=[
            pl.BlockSpec(memory_space=pltpu.VMEM),
            pl.BlockSpec(memory_space=pltpu.VMEM),
        ],
        out_specs=[
            pl.BlockSpec(memory_space=pl.MemorySpace.ANY),
            pl.BlockSpec(memory_space=pltpu.VMEM),
        ],
        out_shape=[
            jax.ShapeDtypeStruct((B, T, F), jnp.float32),
            jax.ShapeDtypeStruct((T, F), jnp.float32),
        ],
        scratch_shapes=[
            pltpu.VMEM((ZROWS, T, F - CP), jnp.float32),
            [pltpu.VMEM((DROWS, T, CP), jnp.float32) for _ in range(NDQ)],
            pltpu.SemaphoreType.DMA((NZQ,)),
            pltpu.SemaphoreType.DMA((NDQ,)),
        ],
    )(attention_mask, x_head)
    return out, att
